# baseline (speedup 1.0000x reference)
import functools
import math

import jax
import jax.numpy as jnp
from jax import lax
from jax.experimental import pallas as pl
from jax.experimental.pallas import tpu as pltpu
from jax.experimental.pallas import tpu_sc as plsc

VOCAB = 1000000
HIDDEN = 64
BATCH = 4096
HIST = 200

NC = 2
NS = 16
NW = NC * NS
LANES = 16

B_TOTAL = BATCH * HIST
B_PER_W = B_TOTAL // NW
CHUNK = 512
NCHUNK = B_PER_W // CHUNK

EMB_SCALE = math.sqrt(HIDDEN)


def _sc_body(idx_hbm, table_hbm, out_hbm, idx_all, rows, gsem):
    wid = lax.axis_index("s") * NC + lax.axis_index("c")
    base = wid * B_PER_W

    pltpu.sync_copy(idx_hbm.at[pl.ds(base, B_PER_W)], idx_all)

    def chunk_step(g, _):
        pltpu.async_copy(
            table_hbm.at[idx_all.at[pl.ds(g * CHUNK, CHUNK)]], rows, gsem
        ).wait()

        def row_step(r, _):
            for c in range(HIDDEN // LANES):
                sl = pl.ds(c * LANES, LANES)
                rows[r, sl] = rows[r, sl] * EMB_SCALE
            return 0

        lax.fori_loop(0, CHUNK, row_step, 0, unroll=2)

        pltpu.sync_copy(rows, out_hbm.at[pl.ds(base + g * CHUNK, CHUNK)])
        return 0

    lax.fori_loop(0, NCHUNK, chunk_step, 0)


@jax.jit
def _emb_lookup(idx_flat, table):
    mesh = plsc.VectorSubcoreMesh(core_axis_name="c", subcore_axis_name="s")
    run = pl.kernel(
        _sc_body,
        out_type=jax.ShapeDtypeStruct((B_TOTAL, HIDDEN), jnp.float32),
        mesh=mesh,
        scratch_types=[
            pltpu.VMEM((B_PER_W,), jnp.int32),
            pltpu.VMEM((CHUNK, HIDDEN), jnp.float32),
            pltpu.SemaphoreType.DMA,
        ],
        compiler_params=pltpu.CompilerParams(use_tc_tiling_on_sc=False),
    )
    return run(idx_flat, table)


def kernel(input_ids, table):
    idx_flat = input_ids.reshape(-1).astype(jnp.int32)
    out = _emb_lookup(idx_flat, table)
    return out.reshape(BATCH, HIST, HIDDEN)
